# baseline (speedup 1.0000x reference)
import functools

import jax
import jax.numpy as jnp
from jax.experimental import pallas as pl
from jax.experimental.pallas import tpu as pltpu

_IN_CAPS = 16
_IN_DIM = 16
_OUT_CAPS = 8
_OUT_DIM = 16
_COUT = _OUT_CAPS * _OUT_DIM
_CIN = _IN_DIM * 4
_H = 32
_HP = _H + 2
_HO = 16
_WO = 16
_NPIX = _HO * _WO
_KW = 9 * _CIN
_K = _KW + 8
_WSRC = _IN_DIM * 4 * 9
_HALF = _IN_CAPS // 2


def _build_weights(w_ref, b_ref):
    wraw = w_ref[...]
    lane = jax.lax.broadcasted_iota(jnp.int32, (_WSRC, _K), 1)
    src = jax.lax.broadcasted_iota(jnp.int32, (_WSRC, _K), 0)
    kh = lane // 192
    kw = (lane // 64) % 3
    cin_dim = (lane % 64) // 4
    rot = lane % 4
    blocks = []
    for s in range(4):
        rot_src = (rot + 4 - s) % 4
        if s == 0:
            khs, kws = kh, kw
        elif s == 1:
            khs, kws = kw, 2 - kh
        elif s == 2:
            khs, kws = 2 - kh, 2 - kw
        else:
            khs, kws = 2 - kw, kh
        src_idx = cin_dim * 36 + rot_src * 9 + khs * 3 + kws
        sel = (src == src_idx) & (lane < _KW)
        p_s = jnp.where(sel, 1.0, 0.0).astype(jnp.float32)
        blocks.append(jax.lax.dot_general(
            wraw, p_s, (((1,), (0,)), ((), ())),
            preferred_element_type=jnp.float32))
    wall = jnp.concatenate(blocks, axis=0)
    eye = jnp.where(
        jax.lax.broadcasted_iota(jnp.int32, (_COUT, _COUT), 0)
        == jax.lax.broadcasted_iota(jnp.int32, (_COUT, _COUT), 1),
        1.0, 0.0).astype(jnp.float32)
    bcol = jax.lax.dot_general(
        eye, b_ref[...], (((1,), (1,)), ((), ())),
        preferred_element_type=jnp.float32)
    lane512 = jax.lax.broadcasted_iota(jnp.int32, (4 * _COUT, _K), 1)
    return jnp.where(lane512 >= _KW,
                     jnp.tile(bcol * 2.0, (4, 1)), wall)


def _conv_squash_body(xa_ref, xb_ref, w_ref, b_ref, o_ref, pad_ref, wmat_ref):
    step = pl.program_id(0)

    @pl.when(step == 0)
    def _prologue():
        pad_ref[...] = jnp.zeros((_HP, _HP, _CIN), jnp.float32)
        wmat_ref[...] = _build_weights(w_ref, b_ref)

    va = xa_ref[0].reshape(_HALF, _CIN, _H, _H)
    vb = xb_ref[0].reshape(_HALF, _CIN, _H, _H)
    xs = jnp.sum(va, axis=0) + jnp.sum(vb, axis=0)
    xs_pm = jnp.transpose(xs, (1, 2, 0))
    pad_ref[1:_H + 1, 1:_H + 1, :] = xs_pm
    y = jnp.sum(wmat_ref[:, _KW:_K], axis=1, keepdims=True)
    for dh in range(3):
        for dw in range(3):
            a = pad_ref[dh:dh + 2 * _HO:2, dw:dw + 2 * _WO:2, :]
            t = dh * 3 + dw
            wt = wmat_ref[:, t * _CIN:(t + 1) * _CIN]
            y = y + jax.lax.dot_general(
                wt, a.reshape(_NPIX, _CIN), (((1,), (1,)), ((), ())),
                preferred_element_type=jnp.float32)
    ys = y.reshape(4, _COUT, _NPIX)
    n2 = jnp.sum(ys * ys, axis=0, keepdims=True)
    norm = jnp.sqrt(n2)
    scale = n2 / (1.0 + n2) / (norm + 1e-8)
    o_ref[0] = jnp.transpose(ys * scale, (1, 0, 2))


@functools.partial(jax.jit, static_argnames=())
def kernel(in_capsules, conv_w, conv_b, ln_gamma, ln_beta):
    del ln_gamma, ln_beta
    nb = in_capsules.shape[0]

    wraw = conv_w.reshape(_COUT, _WSRC)
    brow = conv_b.reshape(1, _COUT)

    out = pl.pallas_call(
        _conv_squash_body,
        grid=(nb,),
        in_specs=[
            pl.BlockSpec((1, _HALF, _IN_DIM, 4, _H, _H),
                         lambda b: (b, 0, 0, 0, 0, 0)),
            pl.BlockSpec((1, _HALF, _IN_DIM, 4, _H, _H),
                         lambda b: (b, 1, 0, 0, 0, 0)),
            pl.BlockSpec((_COUT, _WSRC), lambda b: (0, 0)),
            pl.BlockSpec((1, _COUT), lambda b: (0, 0)),
        ],
        out_specs=pl.BlockSpec((1, _COUT, 4, _NPIX), lambda b: (b, 0, 0, 0)),
        out_shape=jax.ShapeDtypeStruct((nb, _COUT, 4, _NPIX), jnp.float32),
        scratch_shapes=[pltpu.VMEM((_HP, _HP, _CIN), jnp.float32),
                        pltpu.VMEM((4 * _COUT, _K), jnp.float32)],
    )(in_capsules, in_capsules, wraw, brow)

    return out.reshape(nb, _OUT_CAPS, _OUT_DIM, 4, _HO, _WO)


# speedup vs baseline: 1.0152x; 1.0152x over previous
import functools

import jax
import jax.numpy as jnp
from jax.experimental import pallas as pl
from jax.experimental.pallas import tpu as pltpu

_IN_CAPS = 16
_IN_DIM = 16
_OUT_CAPS = 8
_OUT_DIM = 16
_COUT = _OUT_CAPS * _OUT_DIM
_CIN = _IN_DIM * 4
_H = 32
_HP = _H + 2
_HO = 16
_WO = 16
_NPIX = _HO * _WO
_KW = 9 * _CIN
_K = _KW + 8
_WSRC = _IN_DIM * 4 * 9


def _build_weights(w_ref, b_ref):
    wraw = w_ref[...]
    lane = jax.lax.broadcasted_iota(jnp.int32, (_WSRC, _K), 1)
    src = jax.lax.broadcasted_iota(jnp.int32, (_WSRC, _K), 0)
    kh = lane // 192
    kw = (lane // 64) % 3
    cin_dim = (lane % 64) // 4
    rot = lane % 4
    blocks = []
    for s in range(4):
        rot_src = (rot + 4 - s) % 4
        if s == 0:
            khs, kws = kh, kw
        elif s == 1:
            khs, kws = kw, 2 - kh
        elif s == 2:
            khs, kws = 2 - kh, 2 - kw
        else:
            khs, kws = 2 - kw, kh
        src_idx = cin_dim * 36 + rot_src * 9 + khs * 3 + kws
        sel = (src == src_idx) & (lane < _KW)
        p_s = jnp.where(sel, 1.0, 0.0).astype(jnp.float32)
        blocks.append(jax.lax.dot_general(
            wraw, p_s, (((1,), (0,)), ((), ())),
            preferred_element_type=jnp.float32))
    wall = jnp.concatenate(blocks, axis=0)
    eye = jnp.where(
        jax.lax.broadcasted_iota(jnp.int32, (_COUT, _COUT), 0)
        == jax.lax.broadcasted_iota(jnp.int32, (_COUT, _COUT), 1),
        1.0, 0.0).astype(jnp.float32)
    bcol = jax.lax.dot_general(
        eye, b_ref[...], (((1,), (1,)), ((), ())),
        preferred_element_type=jnp.float32)
    lane512 = jax.lax.broadcasted_iota(jnp.int32, (4 * _COUT, _K), 1)
    return jnp.where(lane512 >= _KW,
                     jnp.tile(bcol * 2.0, (4, 1)), wall)


def _conv_squash_body(x_ref, w_ref, b_ref, o_ref, pad_ref, wmat_ref):
    step = pl.program_id(0)

    @pl.when(step == 0)
    def _prologue():
        pad_ref[...] = jnp.zeros((_HP, _HP, _CIN), jnp.float32)
        wmat_ref[...] = _build_weights(w_ref, b_ref)

    v = x_ref[0].reshape(_IN_CAPS, _CIN, _H, _H)
    xs = jnp.sum(v, axis=0)
    xs_pm = jnp.transpose(xs, (1, 2, 0))
    pad_ref[1:_H + 1, 1:_H + 1, :] = xs_pm
    pieces = []
    for dh in range(3):
        for dw in range(3):
            a = pad_ref[dh:dh + 2 * _HO:2, dw:dw + 2 * _WO:2, :]
            pieces.append(a.reshape(_NPIX, _CIN))
    pieces.append(jnp.ones((_NPIX, 8), jnp.float32))
    colt = jnp.concatenate(pieces, axis=1)
    y = jax.lax.dot_general(
        wmat_ref[...], colt, (((1,), (1,)), ((), ())),
        preferred_element_type=jnp.float32)
    ys = y.reshape(4, _COUT, _NPIX)
    n2 = jnp.sum(ys * ys, axis=0, keepdims=True)
    norm = jnp.sqrt(n2)
    scale = n2 / (1.0 + n2) / (norm + 1e-8)
    o_ref[0] = jnp.transpose(ys * scale, (1, 0, 2))


@functools.partial(jax.jit, static_argnames=())
def kernel(in_capsules, conv_w, conv_b, ln_gamma, ln_beta):
    del ln_gamma, ln_beta
    nb = in_capsules.shape[0]

    wraw = conv_w.reshape(_COUT, _WSRC)
    brow = conv_b.reshape(1, _COUT)

    out = pl.pallas_call(
        _conv_squash_body,
        grid=(nb,),
        in_specs=[
            pl.BlockSpec((1, _IN_CAPS, _IN_DIM, 4, _H, _H),
                         lambda b: (b, 0, 0, 0, 0, 0)),
            pl.BlockSpec((_COUT, _WSRC), lambda b: (0, 0)),
            pl.BlockSpec((1, _COUT), lambda b: (0, 0)),
        ],
        out_specs=pl.BlockSpec((1, _COUT, 4, _NPIX), lambda b: (b, 0, 0, 0)),
        out_shape=jax.ShapeDtypeStruct((nb, _COUT, 4, _NPIX), jnp.float32),
        scratch_shapes=[pltpu.VMEM((_HP, _HP, _CIN), jnp.float32),
                        pltpu.VMEM((4 * _COUT, _K), jnp.float32)],
    )(in_capsules, wraw, brow)

    return out.reshape(nb, _OUT_CAPS, _OUT_DIM, 4, _HO, _WO)
